# baseline (speedup 1.0000x reference)
import functools

import jax
import jax.numpy as jnp
from jax import lax
from jax.experimental import pallas as pl
from jax.experimental.pallas import tpu as pltpu
from jax.experimental.pallas import tpu_sc as plsc

N = 10000
E = 320000
D = 128
DG = 16
NC = 2
NS = 16
NW = NC * NS
EPW = E // NW
C = 125
STEPS = EPW // C
SB = 10
NB = STEPS // SB
NPAD = 10112
RPT = NPAD // NS


def _make_segsum(with_deg):
  mesh = plsc.VectorSubcoreMesh(
      core_axis_name="c", subcore_axis_name="s", num_cores=NC, num_subcores=NS)

  acc_t = jax.ShapeDtypeStruct((NC, NPAD, D), jnp.float32)
  out_type = [acc_t, jax.ShapeDtypeStruct((NC, NPAD, DG), jnp.float32)
              ] if with_deg else acc_t
  scratch = [
      pltpu.VMEM((2, SB, C), jnp.int32),
      pltpu.VMEM((2, SB, C), jnp.int32),
      pltpu.VMEM((2, C, D), jnp.float32),
      pltpu.VMEM_SHARED((NPAD, D), jnp.float32),
      pltpu.SemaphoreType.DMA,
      pltpu.SemaphoreType.DMA,
      pltpu.SemaphoreType.DMA,
      pltpu.SemaphoreType.DMA,
      pltpu.SemaphoreType.DMA,
  ]
  if with_deg:
    scratch += [
        pltpu.VMEM((C, DG), jnp.float32),
        pltpu.VMEM_SHARED((NPAD, DG), jnp.float32),
        pltpu.SemaphoreType.DMA,
    ]

  @functools.partial(
      pl.kernel,
      mesh=mesh,
      out_type=out_type,
      scratch_types=scratch,
      compiler_params=pltpu.CompilerParams(use_tc_tiling_on_sc=False),
  )
  def seg(table, srcw, dstw, zeros, *rest):
    if with_deg:
      (ones, zeros_dg, out, out_dg, src_v, dst_v, rows_v, acc_sh,
       gsem0, gsem1, isem, ssem0, ssem1, ones_v, deg_sh, dsem) = rest
    else:
      (out, src_v, dst_v, rows_v, acc_sh,
       gsem0, gsem1, isem, ssem0, ssem1) = rest
    cid = lax.axis_index("c")
    sid = lax.axis_index("s")
    wid = cid * NS + sid
    pltpu.sync_copy(zeros, acc_sh.at[pl.ds(sid * RPT, RPT)])
    if with_deg:
      pltpu.sync_copy(zeros_dg, deg_sh.at[pl.ds(sid * RPT, RPT)])
      pltpu.sync_copy(ones, ones_v)
    pltpu.sync_copy(srcw.at[wid, 0], src_v.at[0])
    pltpu.sync_copy(dstw.at[wid, 0], dst_v.at[0])
    plsc.subcore_barrier()

    pltpu.async_copy(table.at[src_v.at[0, 0]], rows_v.at[0], gsem0)
    pltpu.async_copy(table.at[src_v.at[0, 1]], rows_v.at[1], gsem1)

    def block(b, carry):
      bp = b % 2

      @pl.when(b + 1 < NB)
      def _():
        pltpu.async_copy(srcw.at[wid, b + 1], src_v.at[1 - bp], isem)
        pltpu.async_copy(dstw.at[wid, b + 1], dst_v.at[1 - bp], isem)

      def pair(j, carry2):
        j0 = 2 * j
        j1 = j0 + 1
        pltpu.make_async_copy(table.at[src_v.at[bp, j0]], rows_v.at[0],
                              gsem0).wait()
        pltpu.async_copy(rows_v.at[0], acc_sh.at[dst_v.at[bp, j0]], ssem0,
                         add=True)
        if with_deg:
          pltpu.async_copy(ones_v, deg_sh.at[dst_v.at[bp, j0]], dsem,
                           add=True)
        pltpu.make_async_copy(table.at[src_v.at[bp, j1]], rows_v.at[1],
                              gsem1).wait()
        pltpu.async_copy(rows_v.at[1], acc_sh.at[dst_v.at[bp, j1]], ssem1,
                         add=True)
        if with_deg:
          pltpu.async_copy(ones_v, deg_sh.at[dst_v.at[bp, j1]], dsem,
                           add=True)

        @pl.when(j + 1 < SB // 2)
        def _():
          pltpu.make_async_copy(rows_v.at[0], acc_sh.at[dst_v.at[bp, j0]],
                                ssem0).wait()
          pltpu.async_copy(table.at[src_v.at[bp, j0 + 2]], rows_v.at[0],
                           gsem0)
          pltpu.make_async_copy(rows_v.at[1], acc_sh.at[dst_v.at[bp, j1]],
                                ssem1).wait()
          pltpu.async_copy(table.at[src_v.at[bp, j1 + 2]], rows_v.at[1],
                           gsem1)

        @pl.when((j + 1 >= SB // 2) & (b + 1 < NB))
        def _():
          pltpu.make_async_copy(srcw.at[wid, b + 1], src_v.at[1 - bp],
                                isem).wait()
          pltpu.make_async_copy(dstw.at[wid, b + 1], dst_v.at[1 - bp],
                                isem).wait()
          pltpu.make_async_copy(rows_v.at[0], acc_sh.at[dst_v.at[bp, j0]],
                                ssem0).wait()
          pltpu.async_copy(table.at[src_v.at[1 - bp, 0]], rows_v.at[0], gsem0)
          pltpu.make_async_copy(rows_v.at[1], acc_sh.at[dst_v.at[bp, j1]],
                                ssem1).wait()
          pltpu.async_copy(table.at[src_v.at[1 - bp, 1]], rows_v.at[1], gsem1)

        @pl.when((j + 1 >= SB // 2) & (b + 1 >= NB))
        def _():
          pltpu.make_async_copy(rows_v.at[0], acc_sh.at[dst_v.at[bp, j0]],
                                ssem0).wait()
          pltpu.make_async_copy(rows_v.at[1], acc_sh.at[dst_v.at[bp, j1]],
                                ssem1).wait()

        if with_deg:
          pltpu.make_async_copy(ones_v, deg_sh.at[dst_v.at[bp, j0]],
                                dsem).wait()
          pltpu.make_async_copy(ones_v, deg_sh.at[dst_v.at[bp, j1]],
                                dsem).wait()
        return carry2

      lax.fori_loop(0, SB // 2, pair, 0)
      return carry

    lax.fori_loop(0, NB, block, 0)
    plsc.subcore_barrier()
    pltpu.sync_copy(acc_sh.at[pl.ds(sid * RPT, RPT)],
                    out.at[cid, pl.ds(sid * RPT, RPT)])
    if with_deg:
      pltpu.sync_copy(deg_sh.at[pl.ds(sid * RPT, RPT)],
                      out_dg.at[cid, pl.ds(sid * RPT, RPT)])

  return seg


_segsum_l1 = _make_segsum(True)
_segsum_l2 = _make_segsum(False)

_BN = 1000


def _dense1_body(acc_ref, dg_ref, x_ref, wl_ref, wr_ref, b_ref, h_ref, dv_ref):
  s = acc_ref[0] + acc_ref[1]
  deg = dg_ref[0, :, 0:1] + dg_ref[1, :, 0:1]
  dinv = 1.0 / jnp.maximum(deg, 1.0)
  agg = s * dinv
  h = lax.dot_general(agg, wl_ref[...], (((1,), (1,)), ((), ())),
                      preferred_element_type=jnp.float32)
  h = h + lax.dot_general(x_ref[...], wr_ref[...], (((1,), (1,)), ((), ())),
                          preferred_element_type=jnp.float32)
  h = h + b_ref[...]
  h_ref[...] = jnp.maximum(h, 0.0)
  dv_ref[...] = jnp.broadcast_to(dinv, (_BN, D))


def _dense2_body(acc_ref, dv_ref, h_ref, wl_ref, wr_ref, b_ref, o_ref):
  agg = (acc_ref[0] + acc_ref[1]) * dv_ref[...]
  o = lax.dot_general(agg, wl_ref[...], (((1,), (1,)), ((), ())),
                      preferred_element_type=jnp.float32)
  o = o + lax.dot_general(h_ref[...], wr_ref[...], (((1,), (1,)), ((), ())),
                          preferred_element_type=jnp.float32)
  o_ref[...] = o + b_ref[...]


def _dense1(acc, dg, x, wl, wr, b):
  grid = (N // _BN,)
  return pl.pallas_call(
      _dense1_body,
      grid=grid,
      in_specs=[
          pl.BlockSpec((NC, _BN, D), lambda i: (0, i, 0)),
          pl.BlockSpec((NC, _BN, DG), lambda i: (0, i, 0)),
          pl.BlockSpec((_BN, D), lambda i: (i, 0)),
          pl.BlockSpec((D, D), lambda i: (0, 0)),
          pl.BlockSpec((D, D), lambda i: (0, 0)),
          pl.BlockSpec((1, D), lambda i: (0, 0)),
      ],
      out_specs=[
          pl.BlockSpec((_BN, D), lambda i: (i, 0)),
          pl.BlockSpec((_BN, D), lambda i: (i, 0)),
      ],
      out_shape=[
          jax.ShapeDtypeStruct((N, D), jnp.float32),
          jax.ShapeDtypeStruct((N, D), jnp.float32),
      ],
  )(acc, dg, x, wl, wr, b)


def _dense2(acc, dv, h, wl, wr, b):
  grid = (N // _BN,)
  return pl.pallas_call(
      _dense2_body,
      grid=grid,
      in_specs=[
          pl.BlockSpec((NC, _BN, D), lambda i: (0, i, 0)),
          pl.BlockSpec((_BN, D), lambda i: (i, 0)),
          pl.BlockSpec((_BN, D), lambda i: (i, 0)),
          pl.BlockSpec((D, D), lambda i: (0, 0)),
          pl.BlockSpec((D, D), lambda i: (0, 0)),
          pl.BlockSpec((1, D), lambda i: (0, 0)),
      ],
      out_specs=pl.BlockSpec((_BN, D), lambda i: (i, 0)),
      out_shape=jax.ShapeDtypeStruct((N, D), jnp.float32),
  )(acc, dv, h, wl, wr, b)


def kernel(x, edge_index, W1_l, W1_r, b1, W2_l, W2_r, b2):
  src = edge_index[0].reshape(NW, NB, SB, C)
  dst = edge_index[1].reshape(NW, NB, SB, C)
  zeros_d = jnp.zeros((RPT, D), jnp.float32)
  zeros_dg = jnp.zeros((RPT, DG), jnp.float32)
  ones_c = jnp.ones((C, DG), jnp.float32)

  acc1, dg = _segsum_l1(x, src, dst, zeros_d, ones_c, zeros_dg)
  h, dv = _dense1(acc1, dg, x, W1_l, W1_r, b1.reshape(1, D))
  acc2 = _segsum_l2(h, src, dst, zeros_d)
  out = _dense2(acc2, dv, h, W2_l, W2_r, b2.reshape(1, D))
  return out


# speedup vs baseline: 1.1665x; 1.1665x over previous
import functools

import jax
import jax.numpy as jnp
from jax import lax
from jax.experimental import pallas as pl
from jax.experimental.pallas import tpu as pltpu
from jax.experimental.pallas import tpu_sc as plsc

N = 10000
E = 320000
D = 128
DG = 16
NC = 2
NS = 16
NW = NC * NS
EPW = E // NW
C = 125
STEPS = EPW // C
SB = 10
NB = STEPS // SB
NPAD = 10112
RPT = NPAD // NS


def _make_segsum(with_deg):
  mesh = plsc.VectorSubcoreMesh(
      core_axis_name="c", subcore_axis_name="s", num_cores=NC, num_subcores=NS)

  acc_t = jax.ShapeDtypeStruct((NC, NPAD, D), jnp.float32)
  out_type = [acc_t, jax.ShapeDtypeStruct((NC, NPAD, DG), jnp.float32)
              ] if with_deg else acc_t
  scratch = [
      pltpu.VMEM((2, SB, C), jnp.int32),
      pltpu.VMEM((2, SB, C), jnp.int32),
      pltpu.VMEM((2, C, D), jnp.float32),
      pltpu.VMEM_SHARED((NPAD, D), jnp.float32),
      pltpu.SemaphoreType.DMA,
      pltpu.SemaphoreType.DMA,
      pltpu.SemaphoreType.DMA,
  ]
  if with_deg:
    scratch += [
        pltpu.VMEM((C, DG), jnp.float32),
        pltpu.VMEM_SHARED((NPAD, DG), jnp.float32),
        pltpu.SemaphoreType.DMA,
    ]

  @functools.partial(
      pl.kernel,
      mesh=mesh,
      out_type=out_type,
      scratch_types=scratch,
      compiler_params=pltpu.CompilerParams(use_tc_tiling_on_sc=False),
  )
  def seg(table, srcw, dstw, zeros, *rest):
    if with_deg:
      (ones, zeros_dg, out, out_dg, src_v, dst_v, rows_v, acc_sh,
       gsem0, gsem1, isem, ones_v, deg_sh, dsem) = rest
    else:
      out, src_v, dst_v, rows_v, acc_sh, gsem0, gsem1, isem = rest
    cid = lax.axis_index("c")
    sid = lax.axis_index("s")
    wid = cid * NS + sid
    pltpu.sync_copy(zeros, acc_sh.at[pl.ds(sid * RPT, RPT)])
    if with_deg:
      pltpu.sync_copy(zeros_dg, deg_sh.at[pl.ds(sid * RPT, RPT)])
      pltpu.sync_copy(ones, ones_v)
    pltpu.sync_copy(srcw.at[wid, 0], src_v.at[0])
    pltpu.sync_copy(dstw.at[wid, 0], dst_v.at[0])
    plsc.subcore_barrier()

    pltpu.async_copy(table.at[src_v.at[0, 0]], rows_v.at[0], gsem0)

    def scat(bp, j, buf, sem):
      pltpu.make_async_copy(table.at[src_v.at[bp, j]], rows_v.at[buf],
                            sem).wait()
      pltpu.sync_copy(rows_v.at[buf], acc_sh.at[dst_v.at[bp, j]], add=True)
      if with_deg:
        pltpu.async_copy(ones_v, deg_sh.at[dst_v.at[bp, j]], dsem, add=True)

    def block(b, carry):
      bp = b % 2

      @pl.when(b + 1 < NB)
      def _():
        pltpu.async_copy(srcw.at[wid, b + 1], src_v.at[1 - bp], isem)
        pltpu.async_copy(dstw.at[wid, b + 1], dst_v.at[1 - bp], isem)

      def pair(j, carry2):
        j0 = 2 * j
        j1 = j0 + 1
        pltpu.async_copy(table.at[src_v.at[bp, j1]], rows_v.at[1], gsem1)
        scat(bp, j0, 0, gsem0)

        @pl.when(j1 + 1 < SB)
        def _():
          pltpu.async_copy(table.at[src_v.at[bp, j0 + 2]], rows_v.at[0], gsem0)

        @pl.when((j1 + 1 >= SB) & (b + 1 < NB))
        def _():
          pltpu.make_async_copy(srcw.at[wid, b + 1], src_v.at[1 - bp],
                                isem).wait()
          pltpu.make_async_copy(dstw.at[wid, b + 1], dst_v.at[1 - bp],
                                isem).wait()
          pltpu.async_copy(table.at[src_v.at[1 - bp, 0]], rows_v.at[0], gsem0)

        scat(bp, j1, 1, gsem1)
        if with_deg:
          pltpu.make_async_copy(ones_v, deg_sh.at[dst_v.at[bp, j0]],
                                dsem).wait()
          pltpu.make_async_copy(ones_v, deg_sh.at[dst_v.at[bp, j1]],
                                dsem).wait()
        return carry2

      lax.fori_loop(0, SB // 2, pair, 0)
      return carry

    lax.fori_loop(0, NB, block, 0)
    plsc.subcore_barrier()
    pltpu.sync_copy(acc_sh.at[pl.ds(sid * RPT, RPT)],
                    out.at[cid, pl.ds(sid * RPT, RPT)])
    if with_deg:
      pltpu.sync_copy(deg_sh.at[pl.ds(sid * RPT, RPT)],
                      out_dg.at[cid, pl.ds(sid * RPT, RPT)])

  return seg


_segsum_l1 = _make_segsum(True)
_segsum_l2 = _make_segsum(False)

_BN = 1000


def _dense1_body(acc_ref, dg_ref, x_ref, wl_ref, wr_ref, b_ref, h_ref):
  s = acc_ref[0] + acc_ref[1]
  deg = dg_ref[0, :, 0:1] + dg_ref[1, :, 0:1]
  dinv = 1.0 / jnp.maximum(deg, 1.0)
  agg = s * dinv
  h = lax.dot_general(agg, wl_ref[...], (((1,), (1,)), ((), ())),
                      preferred_element_type=jnp.float32)
  h = h + lax.dot_general(x_ref[...], wr_ref[...], (((1,), (1,)), ((), ())),
                          preferred_element_type=jnp.float32)
  h = h + b_ref[...]
  h_ref[...] = jnp.maximum(h, 0.0)


def _dense2_body(acc_ref, dg_ref, h_ref, wl_ref, wr_ref, b_ref, o_ref):
  deg = dg_ref[0, :, 0:1] + dg_ref[1, :, 0:1]
  dinv = 1.0 / jnp.maximum(deg, 1.0)
  agg = (acc_ref[0] + acc_ref[1]) * dinv
  o = lax.dot_general(agg, wl_ref[...], (((1,), (1,)), ((), ())),
                      preferred_element_type=jnp.float32)
  o = o + lax.dot_general(h_ref[...], wr_ref[...], (((1,), (1,)), ((), ())),
                          preferred_element_type=jnp.float32)
  o_ref[...] = o + b_ref[...]


def _dense1(acc, dg, x, wl, wr, b):
  grid = (N // _BN,)
  return pl.pallas_call(
      _dense1_body,
      grid=grid,
      in_specs=[
          pl.BlockSpec((NC, _BN, D), lambda i: (0, i, 0)),
          pl.BlockSpec((NC, _BN, DG), lambda i: (0, i, 0)),
          pl.BlockSpec((_BN, D), lambda i: (i, 0)),
          pl.BlockSpec((D, D), lambda i: (0, 0)),
          pl.BlockSpec((D, D), lambda i: (0, 0)),
          pl.BlockSpec((1, D), lambda i: (0, 0)),
      ],
      out_specs=pl.BlockSpec((_BN, D), lambda i: (i, 0)),
      out_shape=jax.ShapeDtypeStruct((N, D), jnp.float32),
  )(acc, dg, x, wl, wr, b)


def _dense2(acc, dg, h, wl, wr, b):
  grid = (N // _BN,)
  return pl.pallas_call(
      _dense2_body,
      grid=grid,
      in_specs=[
          pl.BlockSpec((NC, _BN, D), lambda i: (0, i, 0)),
          pl.BlockSpec((NC, _BN, DG), lambda i: (0, i, 0)),
          pl.BlockSpec((_BN, D), lambda i: (i, 0)),
          pl.BlockSpec((D, D), lambda i: (0, 0)),
          pl.BlockSpec((D, D), lambda i: (0, 0)),
          pl.BlockSpec((1, D), lambda i: (0, 0)),
      ],
      out_specs=pl.BlockSpec((_BN, D), lambda i: (i, 0)),
      out_shape=jax.ShapeDtypeStruct((N, D), jnp.float32),
  )(acc, dg, h, wl, wr, b)


def kernel(x, edge_index, W1_l, W1_r, b1, W2_l, W2_r, b2):
  src = edge_index[0].reshape(NW, NB, SB, C)
  dst = edge_index[1].reshape(NW, NB, SB, C)
  zeros_d = jnp.zeros((RPT, D), jnp.float32)
  zeros_dg = jnp.zeros((RPT, DG), jnp.float32)
  ones_c = jnp.ones((C, DG), jnp.float32)

  acc1, dg = _segsum_l1(x, src, dst, zeros_d, ones_c, zeros_dg)
  h = _dense1(acc1, dg, x, W1_l, W1_r, b1.reshape(1, D))
  acc2 = _segsum_l2(h, src, dst, zeros_d)
  out = _dense2(acc2, dg, h, W2_l, W2_r, b2.reshape(1, D))
  return out


# speedup vs baseline: 1.1957x; 1.0251x over previous
import functools

import jax
import jax.numpy as jnp
from jax import lax
from jax.experimental import pallas as pl
from jax.experimental.pallas import tpu as pltpu
from jax.experimental.pallas import tpu_sc as plsc

N = 10000
E = 320000
D = 128
DG = 16
NC = 2
NS = 16
NW = NC * NS
EPW = E // NW
C = 125
STEPS = EPW // C
SB = 10
NB = STEPS // SB
NPAD = 10112
RPT = NPAD // NS


def _make_segsum(with_deg):
  mesh = plsc.VectorSubcoreMesh(
      core_axis_name="c", subcore_axis_name="s", num_cores=NC, num_subcores=NS)

  acc_t = jax.ShapeDtypeStruct((NC, NPAD, D), jnp.float32)
  out_type = [acc_t, jax.ShapeDtypeStruct((NC, NPAD, DG), jnp.float32)
              ] if with_deg else acc_t
  scratch = [
      pltpu.VMEM((2, SB, C), jnp.int32),
      pltpu.VMEM((2, SB, C), jnp.int32),
      pltpu.VMEM((2, C, D), jnp.float32),
      pltpu.VMEM_SHARED((NPAD, D), jnp.float32),
      pltpu.SemaphoreType.DMA,
      pltpu.SemaphoreType.DMA,
      pltpu.SemaphoreType.DMA,
  ]
  if with_deg:
    scratch += [
        pltpu.VMEM((C, DG), jnp.float32),
        pltpu.VMEM_SHARED((NPAD, DG), jnp.float32),
        pltpu.SemaphoreType.DMA,
    ]

  @functools.partial(
      pl.kernel,
      mesh=mesh,
      out_type=out_type,
      scratch_types=scratch,
      compiler_params=pltpu.CompilerParams(use_tc_tiling_on_sc=False),
  )
  def seg(table, srcw, dstw, zeros, *rest):
    if with_deg:
      (ones, zeros_dg, out, out_dg, src_v, dst_v, rows_v, acc_sh,
       gsem0, gsem1, isem, ones_v, deg_sh, dsem) = rest
    else:
      out, src_v, dst_v, rows_v, acc_sh, gsem0, gsem1, isem = rest
    cid = lax.axis_index("c")
    sid = lax.axis_index("s")
    wid = cid * NS + sid
    pltpu.sync_copy(zeros, acc_sh.at[pl.ds(sid * RPT, RPT)])
    if with_deg:
      pltpu.sync_copy(zeros_dg, deg_sh.at[pl.ds(sid * RPT, RPT)])
      pltpu.sync_copy(ones, ones_v)
    pltpu.sync_copy(srcw.at[wid, 0], src_v.at[0])
    pltpu.sync_copy(dstw.at[wid, 0], dst_v.at[0])
    plsc.subcore_barrier()

    pltpu.async_copy(table.at[src_v.at[0, 0]], rows_v.at[0], gsem0)

    def scat(bp, j, buf, sem):
      pltpu.make_async_copy(table.at[src_v.at[bp, j]], rows_v.at[buf],
                            sem).wait()
      pltpu.sync_copy(rows_v.at[buf], acc_sh.at[dst_v.at[bp, j]], add=True)
      if with_deg:
        pltpu.async_copy(ones_v, deg_sh.at[dst_v.at[bp, j]], dsem, add=True)

    def drain_deg(n):
      def d(s, carry):
        pltpu.make_async_copy(ones_v, deg_sh.at[dst_v.at[0, 0]], dsem).wait()
        return carry
      lax.fori_loop(0, n, d, 0)

    def block(b, carry):
      bp = b % 2
      if with_deg:
        @pl.when(b > 0)
        def _():
          drain_deg(SB)

      @pl.when(b + 1 < NB)
      def _():
        pltpu.async_copy(srcw.at[wid, b + 1], src_v.at[1 - bp], isem)
        pltpu.async_copy(dstw.at[wid, b + 1], dst_v.at[1 - bp], isem)

      def pair(j, carry2):
        j0 = 2 * j
        j1 = j0 + 1
        pltpu.async_copy(table.at[src_v.at[bp, j1]], rows_v.at[1], gsem1)
        scat(bp, j0, 0, gsem0)

        @pl.when(j1 + 1 < SB)
        def _():
          pltpu.async_copy(table.at[src_v.at[bp, j0 + 2]], rows_v.at[0], gsem0)

        @pl.when((j1 + 1 >= SB) & (b + 1 < NB))
        def _():
          pltpu.make_async_copy(srcw.at[wid, b + 1], src_v.at[1 - bp],
                                isem).wait()
          pltpu.make_async_copy(dstw.at[wid, b + 1], dst_v.at[1 - bp],
                                isem).wait()
          pltpu.async_copy(table.at[src_v.at[1 - bp, 0]], rows_v.at[0], gsem0)

        scat(bp, j1, 1, gsem1)
        return carry2

      lax.fori_loop(0, SB // 2, pair, 0)
      return carry

    lax.fori_loop(0, NB, block, 0)
    if with_deg:
      drain_deg(SB)
    plsc.subcore_barrier()
    pltpu.sync_copy(acc_sh.at[pl.ds(sid * RPT, RPT)],
                    out.at[cid, pl.ds(sid * RPT, RPT)])
    if with_deg:
      pltpu.sync_copy(deg_sh.at[pl.ds(sid * RPT, RPT)],
                      out_dg.at[cid, pl.ds(sid * RPT, RPT)])

  return seg


_segsum_l1 = _make_segsum(True)
_segsum_l2 = _make_segsum(False)

_BN = 1000


def _dense1_body(acc_ref, dg_ref, x_ref, wl_ref, wr_ref, b_ref, h_ref):
  s = acc_ref[0] + acc_ref[1]
  deg = dg_ref[0, :, 0:1] + dg_ref[1, :, 0:1]
  dinv = 1.0 / jnp.maximum(deg, 1.0)
  agg = s * dinv
  h = lax.dot_general(agg, wl_ref[...], (((1,), (1,)), ((), ())),
                      preferred_element_type=jnp.float32)
  h = h + lax.dot_general(x_ref[...], wr_ref[...], (((1,), (1,)), ((), ())),
                          preferred_element_type=jnp.float32)
  h = h + b_ref[...]
  h_ref[...] = jnp.maximum(h, 0.0)


def _dense2_body(acc_ref, dg_ref, h_ref, wl_ref, wr_ref, b_ref, o_ref):
  deg = dg_ref[0, :, 0:1] + dg_ref[1, :, 0:1]
  dinv = 1.0 / jnp.maximum(deg, 1.0)
  agg = (acc_ref[0] + acc_ref[1]) * dinv
  o = lax.dot_general(agg, wl_ref[...], (((1,), (1,)), ((), ())),
                      preferred_element_type=jnp.float32)
  o = o + lax.dot_general(h_ref[...], wr_ref[...], (((1,), (1,)), ((), ())),
                          preferred_element_type=jnp.float32)
  o_ref[...] = o + b_ref[...]


def _dense1(acc, dg, x, wl, wr, b):
  grid = (N // _BN,)
  return pl.pallas_call(
      _dense1_body,
      grid=grid,
      in_specs=[
          pl.BlockSpec((NC, _BN, D), lambda i: (0, i, 0)),
          pl.BlockSpec((NC, _BN, DG), lambda i: (0, i, 0)),
          pl.BlockSpec((_BN, D), lambda i: (i, 0)),
          pl.BlockSpec((D, D), lambda i: (0, 0)),
          pl.BlockSpec((D, D), lambda i: (0, 0)),
          pl.BlockSpec((1, D), lambda i: (0, 0)),
      ],
      out_specs=pl.BlockSpec((_BN, D), lambda i: (i, 0)),
      out_shape=jax.ShapeDtypeStruct((N, D), jnp.float32),
  )(acc, dg, x, wl, wr, b)


def _dense2(acc, dg, h, wl, wr, b):
  grid = (N // _BN,)
  return pl.pallas_call(
      _dense2_body,
      grid=grid,
      in_specs=[
          pl.BlockSpec((NC, _BN, D), lambda i: (0, i, 0)),
          pl.BlockSpec((NC, _BN, DG), lambda i: (0, i, 0)),
          pl.BlockSpec((_BN, D), lambda i: (i, 0)),
          pl.BlockSpec((D, D), lambda i: (0, 0)),
          pl.BlockSpec((D, D), lambda i: (0, 0)),
          pl.BlockSpec((1, D), lambda i: (0, 0)),
      ],
      out_specs=pl.BlockSpec((_BN, D), lambda i: (i, 0)),
      out_shape=jax.ShapeDtypeStruct((N, D), jnp.float32),
  )(acc, dg, h, wl, wr, b)


def kernel(x, edge_index, W1_l, W1_r, b1, W2_l, W2_r, b2):
  src = edge_index[0].reshape(NW, NB, SB, C)
  dst = edge_index[1].reshape(NW, NB, SB, C)
  zeros_d = jnp.zeros((RPT, D), jnp.float32)
  zeros_dg = jnp.zeros((RPT, DG), jnp.float32)
  ones_c = jnp.ones((C, DG), jnp.float32)

  acc1, dg = _segsum_l1(x, src, dst, zeros_d, ones_c, zeros_dg)
  h = _dense1(acc1, dg, x, W1_l, W1_r, b1.reshape(1, D))
  acc2 = _segsum_l2(h, src, dst, zeros_d)
  out = _dense2(acc2, dg, h, W2_l, W2_r, b2.reshape(1, D))
  return out


# speedup vs baseline: 1.2687x; 1.0611x over previous
import functools

import jax
import jax.numpy as jnp
import numpy as np
from jax import lax
from jax.experimental import pallas as pl
from jax.experimental.pallas import tpu as pltpu
from jax.experimental.pallas import tpu_sc as plsc

N = 10000
E = 320000
D = 128
DG = 16
NC = 2
NS = 16
NW = NC * NS
EPW = E // NW
C = 125
STEPS = EPW // C
SB = 10
NB = STEPS // SB
NPAD = 10112
RPT = NPAD // NS


def _make_segsum(with_deg):
  mesh = plsc.VectorSubcoreMesh(
      core_axis_name="c", subcore_axis_name="s", num_cores=NC, num_subcores=NS)

  acc_t = jax.ShapeDtypeStruct((NC, NPAD, D), jnp.float32)
  out_type = [acc_t, jax.ShapeDtypeStruct((NC, NPAD, DG), jnp.float32)
              ] if with_deg else acc_t
  scratch = [
      pltpu.VMEM((2, SB, C), jnp.int32),
      pltpu.VMEM((2, SB, C), jnp.int32),
      pltpu.VMEM((2, C, D), jnp.float32),
      pltpu.VMEM_SHARED((NPAD, D), jnp.float32),
      pltpu.SemaphoreType.DMA,
      pltpu.SemaphoreType.DMA,
      pltpu.SemaphoreType.DMA,
  ]
  if with_deg:
    scratch += [
        pltpu.VMEM((C, DG), jnp.float32),
        pltpu.VMEM_SHARED((NPAD, DG), jnp.float32),
        pltpu.SemaphoreType.DMA,
    ]

  @functools.partial(
      pl.kernel,
      mesh=mesh,
      out_type=out_type,
      scratch_types=scratch,
      compiler_params=pltpu.CompilerParams(use_tc_tiling_on_sc=False),
  )
  def seg(table, idx5, zeros, *rest):
    if with_deg:
      (ones, zeros_dg, out, out_dg, src_v, dst_v, rows_v, acc_sh,
       gsem0, gsem1, isem, ones_v, deg_sh, dsem) = rest
    else:
      out, src_v, dst_v, rows_v, acc_sh, gsem0, gsem1, isem = rest
    cid = lax.axis_index("c")
    sid = lax.axis_index("s")
    wid = cid * NS + sid
    pltpu.sync_copy(zeros, acc_sh.at[pl.ds(sid * RPT, RPT)])
    if with_deg:
      pltpu.sync_copy(zeros_dg, deg_sh.at[pl.ds(sid * RPT, RPT)])
      pltpu.sync_copy(ones, ones_v)
    pltpu.sync_copy(idx5.at[0, wid, 0], src_v.at[0])
    pltpu.sync_copy(idx5.at[1, wid, 0], dst_v.at[0])
    plsc.subcore_barrier()

    pltpu.async_copy(table.at[src_v.at[0, 0]], rows_v.at[0], gsem0)

    def scat(bp, j, buf, sem):
      pltpu.make_async_copy(table.at[src_v.at[bp, j]], rows_v.at[buf],
                            sem).wait()
      pltpu.sync_copy(rows_v.at[buf], acc_sh.at[dst_v.at[bp, j]], add=True)
      if with_deg:
        pltpu.async_copy(ones_v, deg_sh.at[dst_v.at[bp, j]], dsem, add=True)

    def drain_deg(n):
      def d(s, carry):
        pltpu.make_async_copy(ones_v, deg_sh.at[dst_v.at[0, 0]], dsem).wait()
        return carry
      lax.fori_loop(0, n, d, 0)

    def block(b, carry):
      bp = b % 2
      if with_deg:
        @pl.when(b > 0)
        def _():
          drain_deg(SB)

      @pl.when(b + 1 < NB)
      def _():
        pltpu.async_copy(idx5.at[0, wid, b + 1], src_v.at[1 - bp], isem)
        pltpu.async_copy(idx5.at[1, wid, b + 1], dst_v.at[1 - bp], isem)

      def pair(j, carry2):
        j0 = 2 * j
        j1 = j0 + 1
        pltpu.async_copy(table.at[src_v.at[bp, j1]], rows_v.at[1], gsem1)
        scat(bp, j0, 0, gsem0)

        @pl.when(j1 + 1 < SB)
        def _():
          pltpu.async_copy(table.at[src_v.at[bp, j0 + 2]], rows_v.at[0], gsem0)

        @pl.when((j1 + 1 >= SB) & (b + 1 < NB))
        def _():
          pltpu.make_async_copy(idx5.at[0, wid, b + 1], src_v.at[1 - bp],
                                isem).wait()
          pltpu.make_async_copy(idx5.at[1, wid, b + 1], dst_v.at[1 - bp],
                                isem).wait()
          pltpu.async_copy(table.at[src_v.at[1 - bp, 0]], rows_v.at[0], gsem0)

        scat(bp, j1, 1, gsem1)
        return carry2

      lax.fori_loop(0, SB // 2, pair, 0)
      return carry

    lax.fori_loop(0, NB, block, 0)
    if with_deg:
      drain_deg(SB)
    plsc.subcore_barrier()
    pltpu.sync_copy(acc_sh.at[pl.ds(sid * RPT, RPT)],
                    out.at[cid, pl.ds(sid * RPT, RPT)])
    if with_deg:
      pltpu.sync_copy(deg_sh.at[pl.ds(sid * RPT, RPT)],
                      out_dg.at[cid, pl.ds(sid * RPT, RPT)])

  return seg


_segsum_l1 = _make_segsum(True)
_segsum_l2 = _make_segsum(False)

_BN = 1024


def _deg_col(dg_ref):
  d = dg_ref[0] + dg_ref[1]
  dd = jnp.reshape(jnp.broadcast_to(d[:, None, :], (_BN // 8, 8, D)),
                   (_BN, D))
  rows = lax.broadcasted_iota(jnp.int32, (_BN, D), 0)
  lanes = lax.broadcasted_iota(jnp.int32, (_BN, D), 1)
  sel = lanes == (rows % 8) * DG
  return jnp.sum(jnp.where(sel, dd, 0.0), axis=1, keepdims=True)


def _dense1_body(acc_ref, dg_ref, x_ref, wl_ref, wr_ref, b_ref, h_ref):
  s = acc_ref[0] + acc_ref[1]
  dinv = 1.0 / jnp.maximum(_deg_col(dg_ref), 1.0)
  agg = s * dinv
  h = lax.dot_general(agg, wl_ref[...], (((1,), (1,)), ((), ())),
                      preferred_element_type=jnp.float32)
  h = h + lax.dot_general(x_ref[...], wr_ref[...], (((1,), (1,)), ((), ())),
                          preferred_element_type=jnp.float32)
  h = h + b_ref[...]
  h_ref[...] = jnp.maximum(h, 0.0)


def _dense2_body(acc_ref, dg_ref, h_ref, wl_ref, wr_ref, b_ref, o_ref):
  dinv = 1.0 / jnp.maximum(_deg_col(dg_ref), 1.0)
  agg = (acc_ref[0] + acc_ref[1]) * dinv
  o = lax.dot_general(agg, wl_ref[...], (((1,), (1,)), ((), ())),
                      preferred_element_type=jnp.float32)
  o = o + lax.dot_general(h_ref[...], wr_ref[...], (((1,), (1,)), ((), ())),
                          preferred_element_type=jnp.float32)
  o_ref[...] = o + b_ref[...]


def _dense1(acc, dg, x, wl, wr, b):
  grid = (pl.cdiv(N, _BN),)
  return pl.pallas_call(
      _dense1_body,
      grid=grid,
      in_specs=[
          pl.BlockSpec((NC, _BN, D), lambda i: (0, i, 0)),
          pl.BlockSpec((NC, _BN // 8, D), lambda i: (0, i, 0)),
          pl.BlockSpec((_BN, D), lambda i: (i, 0)),
          pl.BlockSpec((D, D), lambda i: (0, 0)),
          pl.BlockSpec((D, D), lambda i: (0, 0)),
          pl.BlockSpec((1, D), lambda i: (0, 0)),
      ],
      out_specs=pl.BlockSpec((_BN, D), lambda i: (i, 0)),
      out_shape=jax.ShapeDtypeStruct((N, D), jnp.float32),
  )(acc, dg, x, wl, wr, b)


def _dense2(acc, dg, h, wl, wr, b):
  grid = (pl.cdiv(N, _BN),)
  return pl.pallas_call(
      _dense2_body,
      grid=grid,
      in_specs=[
          pl.BlockSpec((NC, _BN, D), lambda i: (0, i, 0)),
          pl.BlockSpec((NC, _BN // 8, D), lambda i: (0, i, 0)),
          pl.BlockSpec((_BN, D), lambda i: (i, 0)),
          pl.BlockSpec((D, D), lambda i: (0, 0)),
          pl.BlockSpec((D, D), lambda i: (0, 0)),
          pl.BlockSpec((1, D), lambda i: (0, 0)),
      ],
      out_specs=pl.BlockSpec((_BN, D), lambda i: (i, 0)),
      out_shape=jax.ShapeDtypeStruct((N, D), jnp.float32),
  )(acc, dg, h, wl, wr, b)


_ZEROS_D = np.zeros((RPT, D), np.float32)
_ZEROS_DG = np.zeros((RPT, DG), np.float32)
_ONES_C = np.ones((C, DG), np.float32)


def kernel(x, edge_index, W1_l, W1_r, b1, W2_l, W2_r, b2):
  idx5 = edge_index.reshape(2, NW, NB, SB, C)
  acc1, dg = _segsum_l1(x, idx5, _ZEROS_D, _ONES_C, _ZEROS_DG)
  dgb = dg.reshape(NC, NPAD // 8, 8 * DG)
  h = _dense1(acc1, dgb, x, W1_l, W1_r, b1.reshape(1, D))
  acc2 = _segsum_l2(h, idx5, _ZEROS_D)
  out = _dense2(acc2, dgb, h, W2_l, W2_r, b2.reshape(1, D))
  return out


# speedup vs baseline: 1.2844x; 1.0123x over previous
import functools

import jax
import jax.numpy as jnp
import numpy as np
from jax import lax
from jax.experimental import pallas as pl
from jax.experimental.pallas import tpu as pltpu
from jax.experimental.pallas import tpu_sc as plsc

N = 10000
E = 320000
D = 128
DG = 16
NC = 2
NS = 16
NW = NC * NS
EPW = E // NW
C = 125
STEPS = EPW // C
SB = 10
NB = STEPS // SB
NPAD = 10112
RPT = NPAD // NS


def _make_segsum(with_deg):
  mesh = plsc.VectorSubcoreMesh(
      core_axis_name="c", subcore_axis_name="s", num_cores=NC, num_subcores=NS)

  acc_t = jax.ShapeDtypeStruct((NC, NPAD, D), jnp.float32)
  out_type = [acc_t, jax.ShapeDtypeStruct((NC, NPAD, DG), jnp.float32)
              ] if with_deg else acc_t
  scratch = [
      pltpu.VMEM((2, SB, C), jnp.int32),
      pltpu.VMEM((2, SB, C), jnp.int32),
      pltpu.VMEM((2, C, D), jnp.float32),
      pltpu.VMEM_SHARED((NPAD, D), jnp.float32),
      pltpu.SemaphoreType.DMA,
      pltpu.SemaphoreType.DMA,
      pltpu.SemaphoreType.DMA,
  ]
  if with_deg:
    scratch += [
        pltpu.VMEM((C, DG), jnp.float32),
        pltpu.VMEM_SHARED((NPAD, DG), jnp.float32),
        pltpu.SemaphoreType.DMA,
    ]

  @functools.partial(
      pl.kernel,
      mesh=mesh,
      out_type=out_type,
      scratch_types=scratch,
      compiler_params=pltpu.CompilerParams(use_tc_tiling_on_sc=False),
  )
  def seg(table, idx5, zeros, *rest):
    if with_deg:
      (ones, zeros_dg, out, out_dg, src_v, dst_v, rows_v, acc_sh,
       gsem0, gsem1, isem, ones_v, deg_sh, dsem) = rest
    else:
      out, src_v, dst_v, rows_v, acc_sh, gsem0, gsem1, isem = rest
    cid = lax.axis_index("c")
    sid = lax.axis_index("s")
    wid = cid * NS + sid
    pltpu.sync_copy(zeros, acc_sh.at[pl.ds(sid * RPT, RPT)])
    if with_deg:
      pltpu.sync_copy(zeros_dg, deg_sh.at[pl.ds(sid * RPT, RPT)])
      pltpu.sync_copy(ones, ones_v)
    pltpu.sync_copy(idx5.at[0, wid, 0], src_v.at[0])
    pltpu.sync_copy(idx5.at[1, wid, 0], dst_v.at[0])
    plsc.subcore_barrier()

    pltpu.async_copy(table.at[src_v.at[0, 0]], rows_v.at[0], gsem0)

    def scat(bp, j, buf, sem):
      pltpu.make_async_copy(table.at[src_v.at[bp, j]], rows_v.at[buf],
                            sem).wait()
      pltpu.sync_copy(rows_v.at[buf], acc_sh.at[dst_v.at[bp, j]], add=True)
      if with_deg:
        pltpu.async_copy(ones_v, deg_sh.at[dst_v.at[bp, j]], dsem, add=True)

    def drain_deg(n):
      def d(s, carry):
        pltpu.make_async_copy(ones_v, deg_sh.at[dst_v.at[0, 0]], dsem).wait()
        return carry
      lax.fori_loop(0, n, d, 0)

    def block(b, carry):
      bp = b % 2
      if with_deg:
        @pl.when(b > 0)
        def _():
          drain_deg(SB)

      @pl.when(b + 1 < NB)
      def _():
        pltpu.async_copy(idx5.at[0, wid, b + 1], src_v.at[1 - bp], isem)
        pltpu.async_copy(idx5.at[1, wid, b + 1], dst_v.at[1 - bp], isem)

      def pair(j, carry2):
        j0 = 2 * j
        j1 = j0 + 1
        pltpu.async_copy(table.at[src_v.at[bp, j1]], rows_v.at[1], gsem1)
        scat(bp, j0, 0, gsem0)

        @pl.when(j1 + 1 < SB)
        def _():
          pltpu.async_copy(table.at[src_v.at[bp, j0 + 2]], rows_v.at[0], gsem0)

        @pl.when((j1 + 1 >= SB) & (b + 1 < NB))
        def _():
          pltpu.make_async_copy(idx5.at[0, wid, b + 1], src_v.at[1 - bp],
                                isem).wait()
          pltpu.make_async_copy(idx5.at[1, wid, b + 1], dst_v.at[1 - bp],
                                isem).wait()
          pltpu.async_copy(table.at[src_v.at[1 - bp, 0]], rows_v.at[0], gsem0)

        scat(bp, j1, 1, gsem1)
        return carry2

      lax.fori_loop(0, SB // 2, pair, 0)
      return carry

    lax.fori_loop(0, NB, block, 0)
    if with_deg:
      drain_deg(SB)
    plsc.subcore_barrier()
    pltpu.sync_copy(acc_sh.at[pl.ds(sid * RPT, RPT)],
                    out.at[cid, pl.ds(sid * RPT, RPT)])
    if with_deg:
      pltpu.sync_copy(deg_sh.at[pl.ds(sid * RPT, RPT)],
                      out_dg.at[cid, pl.ds(sid * RPT, RPT)])

  return seg


_segsum_l1 = _make_segsum(True)
_segsum_l2 = _make_segsum(False)

_BN = 2048


def _deg_col(dg_ref):
  d = dg_ref[0] + dg_ref[1]
  dd = jnp.reshape(jnp.broadcast_to(d[:, None, :], (_BN // 8, 8, D)),
                   (_BN, D))
  rows = lax.broadcasted_iota(jnp.int32, (_BN, D), 0)
  lanes = lax.broadcasted_iota(jnp.int32, (_BN, D), 1)
  sel = lanes == (rows % 8) * DG
  return jnp.sum(jnp.where(sel, dd, 0.0), axis=1, keepdims=True)


def _dense1_body(acc_ref, dg_ref, x_ref, wl_ref, wr_ref, b_ref, h_ref):
  s = acc_ref[0] + acc_ref[1]
  dinv = 1.0 / jnp.maximum(_deg_col(dg_ref), 1.0)
  agg = s * dinv
  h = lax.dot_general(agg, wl_ref[...], (((1,), (1,)), ((), ())),
                      preferred_element_type=jnp.float32)
  h = h + lax.dot_general(x_ref[...], wr_ref[...], (((1,), (1,)), ((), ())),
                          preferred_element_type=jnp.float32)
  h = h + b_ref[...]
  h_ref[...] = jnp.maximum(h, 0.0)


def _dense2_body(acc_ref, dg_ref, h_ref, wl_ref, wr_ref, b_ref, o_ref):
  dinv = 1.0 / jnp.maximum(_deg_col(dg_ref), 1.0)
  agg = (acc_ref[0] + acc_ref[1]) * dinv
  o = lax.dot_general(agg, wl_ref[...], (((1,), (1,)), ((), ())),
                      preferred_element_type=jnp.float32)
  o = o + lax.dot_general(h_ref[...], wr_ref[...], (((1,), (1,)), ((), ())),
                          preferred_element_type=jnp.float32)
  o_ref[...] = o + b_ref[...]


def _dense1(acc, dg, x, wl, wr, b):
  grid = (pl.cdiv(N, _BN),)
  return pl.pallas_call(
      _dense1_body,
      grid=grid,
      in_specs=[
          pl.BlockSpec((NC, _BN, D), lambda i: (0, i, 0)),
          pl.BlockSpec((NC, _BN // 8, D), lambda i: (0, i, 0)),
          pl.BlockSpec((_BN, D), lambda i: (i, 0)),
          pl.BlockSpec((D, D), lambda i: (0, 0)),
          pl.BlockSpec((D, D), lambda i: (0, 0)),
          pl.BlockSpec((1, D), lambda i: (0, 0)),
      ],
      out_specs=pl.BlockSpec((_BN, D), lambda i: (i, 0)),
      out_shape=jax.ShapeDtypeStruct((N, D), jnp.float32),
  )(acc, dg, x, wl, wr, b)


def _dense2(acc, dg, h, wl, wr, b):
  grid = (pl.cdiv(N, _BN),)
  return pl.pallas_call(
      _dense2_body,
      grid=grid,
      in_specs=[
          pl.BlockSpec((NC, _BN, D), lambda i: (0, i, 0)),
          pl.BlockSpec((NC, _BN // 8, D), lambda i: (0, i, 0)),
          pl.BlockSpec((_BN, D), lambda i: (i, 0)),
          pl.BlockSpec((D, D), lambda i: (0, 0)),
          pl.BlockSpec((D, D), lambda i: (0, 0)),
          pl.BlockSpec((1, D), lambda i: (0, 0)),
      ],
      out_specs=pl.BlockSpec((_BN, D), lambda i: (i, 0)),
      out_shape=jax.ShapeDtypeStruct((N, D), jnp.float32),
  )(acc, dg, h, wl, wr, b)


_ZEROS_D = np.zeros((RPT, D), np.float32)
_ZEROS_DG = np.zeros((RPT, DG), np.float32)
_ONES_C = np.ones((C, DG), np.float32)


def kernel(x, edge_index, W1_l, W1_r, b1, W2_l, W2_r, b2):
  idx5 = edge_index.reshape(2, NW, NB, SB, C)
  acc1, dg = _segsum_l1(x, idx5, _ZEROS_D, _ONES_C, _ZEROS_DG)
  dgb = dg.reshape(NC, NPAD // 8, 8 * DG)
  h = _dense1(acc1, dgb, x, W1_l, W1_r, b1.reshape(1, D))
  acc2 = _segsum_l2(h, idx5, _ZEROS_D)
  out = _dense2(acc2, dgb, h, W2_l, W2_r, b2.reshape(1, D))
  return out


# speedup vs baseline: 1.2929x; 1.0066x over previous
import functools

import jax
import jax.numpy as jnp
import numpy as np
from jax import lax
from jax.experimental import pallas as pl
from jax.experimental.pallas import tpu as pltpu
from jax.experimental.pallas import tpu_sc as plsc

N = 10000
E = 320000
D = 128
DG = 16
NC = 2
NS = 16
NW = NC * NS
EPW = E // NW
C = 125
STEPS = EPW // C
SB = 10
NB = STEPS // SB
NPAD = 10112
RPT = NPAD // NS


def _make_segsum(with_deg):
  mesh = plsc.VectorSubcoreMesh(
      core_axis_name="c", subcore_axis_name="s", num_cores=NC, num_subcores=NS)

  acc_t = jax.ShapeDtypeStruct((NC, NPAD, D), jnp.float32)
  out_type = [acc_t, jax.ShapeDtypeStruct((NC, NPAD, DG), jnp.float32)
              ] if with_deg else acc_t
  scratch = [
      pltpu.VMEM((2, SB, C), jnp.int32),
      pltpu.VMEM((2, SB, C), jnp.int32),
      pltpu.VMEM((2, C, D), jnp.float32),
      pltpu.VMEM_SHARED((NPAD, D), jnp.float32),
      pltpu.SemaphoreType.DMA,
      pltpu.SemaphoreType.DMA,
      pltpu.SemaphoreType.DMA,
  ]
  if with_deg:
    scratch += [
        pltpu.VMEM((C, DG), jnp.float32),
        pltpu.VMEM_SHARED((NPAD, DG), jnp.float32),
        pltpu.SemaphoreType.DMA,
    ]

  @functools.partial(
      pl.kernel,
      mesh=mesh,
      out_type=out_type,
      scratch_types=scratch,
      compiler_params=pltpu.CompilerParams(use_tc_tiling_on_sc=False),
  )
  def seg(table, idx5, zeros, *rest):
    if with_deg:
      (ones, zeros_dg, out, out_dg, src_v, dst_v, rows_v, acc_sh,
       gsem0, gsem1, isem, ones_v, deg_sh, dsem) = rest
    else:
      out, src_v, dst_v, rows_v, acc_sh, gsem0, gsem1, isem = rest
    cid = lax.axis_index("c")
    sid = lax.axis_index("s")
    wid = cid * NS + sid
    pltpu.sync_copy(zeros, acc_sh.at[pl.ds(sid * RPT, RPT)])
    if with_deg:
      pltpu.sync_copy(zeros_dg, deg_sh.at[pl.ds(sid * RPT, RPT)])
      pltpu.sync_copy(ones, ones_v)
    pltpu.sync_copy(idx5.at[0, wid, 0], src_v.at[0])
    pltpu.sync_copy(idx5.at[1, wid, 0], dst_v.at[0])
    plsc.subcore_barrier()

    pltpu.async_copy(table.at[src_v.at[0, 0]], rows_v.at[0], gsem0)

    def scat(bp, j, buf, sem):
      pltpu.make_async_copy(table.at[src_v.at[bp, j]], rows_v.at[buf],
                            sem).wait()
      pltpu.sync_copy(rows_v.at[buf], acc_sh.at[dst_v.at[bp, j]], add=True)
      if with_deg:
        pltpu.async_copy(ones_v, deg_sh.at[dst_v.at[bp, j]], dsem, add=True)

    def drain_deg(n):
      def d(s, carry):
        pltpu.make_async_copy(ones_v, deg_sh.at[dst_v.at[0, 0]], dsem).wait()
        return carry
      lax.fori_loop(0, n, d, 0)

    def block(b, carry):
      bp = b % 2
      if with_deg:
        @pl.when(b > 0)
        def _():
          drain_deg(SB)

      @pl.when(b + 1 < NB)
      def _():
        pltpu.async_copy(idx5.at[0, wid, b + 1], src_v.at[1 - bp], isem)
        pltpu.async_copy(idx5.at[1, wid, b + 1], dst_v.at[1 - bp], isem)

      def pair(j, carry2):
        j0 = 2 * j
        j1 = j0 + 1
        pltpu.async_copy(table.at[src_v.at[bp, j1]], rows_v.at[1], gsem1)
        scat(bp, j0, 0, gsem0)

        @pl.when(j1 + 1 < SB)
        def _():
          pltpu.async_copy(table.at[src_v.at[bp, j0 + 2]], rows_v.at[0], gsem0)

        @pl.when((j1 + 1 >= SB) & (b + 1 < NB))
        def _():
          pltpu.make_async_copy(idx5.at[0, wid, b + 1], src_v.at[1 - bp],
                                isem).wait()
          pltpu.make_async_copy(idx5.at[1, wid, b + 1], dst_v.at[1 - bp],
                                isem).wait()
          pltpu.async_copy(table.at[src_v.at[1 - bp, 0]], rows_v.at[0], gsem0)

        scat(bp, j1, 1, gsem1)
        return carry2

      lax.fori_loop(0, SB // 2, pair, 0)
      return carry

    lax.fori_loop(0, NB, block, 0)
    if with_deg:
      drain_deg(SB)
    plsc.subcore_barrier()
    pltpu.sync_copy(acc_sh.at[pl.ds(sid * RPT, RPT)],
                    out.at[cid, pl.ds(sid * RPT, RPT)])
    if with_deg:
      pltpu.sync_copy(deg_sh.at[pl.ds(sid * RPT, RPT)],
                      out_dg.at[cid, pl.ds(sid * RPT, RPT)])

  return seg


_segsum_l1 = _make_segsum(True)
_segsum_l2 = _make_segsum(False)

_BN = 4096


def _deg_col(dg_ref):
  d = dg_ref[0] + dg_ref[1]
  dd = jnp.reshape(jnp.broadcast_to(d[:, None, :], (_BN // 8, 8, D)),
                   (_BN, D))
  rows = lax.broadcasted_iota(jnp.int32, (_BN, D), 0)
  lanes = lax.broadcasted_iota(jnp.int32, (_BN, D), 1)
  sel = lanes == (rows % 8) * DG
  return jnp.sum(jnp.where(sel, dd, 0.0), axis=1, keepdims=True)


def _dense1_body(acc_ref, dg_ref, x_ref, wl_ref, wr_ref, b_ref, h_ref):
  s = acc_ref[0] + acc_ref[1]
  dinv = 1.0 / jnp.maximum(_deg_col(dg_ref), 1.0)
  agg = s * dinv
  h = lax.dot_general(agg, wl_ref[...], (((1,), (1,)), ((), ())),
                      preferred_element_type=jnp.float32)
  h = h + lax.dot_general(x_ref[...], wr_ref[...], (((1,), (1,)), ((), ())),
                          preferred_element_type=jnp.float32)
  h = h + b_ref[...]
  h_ref[...] = jnp.maximum(h, 0.0)


def _dense2_body(acc_ref, dg_ref, h_ref, wl_ref, wr_ref, b_ref, o_ref):
  dinv = 1.0 / jnp.maximum(_deg_col(dg_ref), 1.0)
  agg = (acc_ref[0] + acc_ref[1]) * dinv
  o = lax.dot_general(agg, wl_ref[...], (((1,), (1,)), ((), ())),
                      preferred_element_type=jnp.float32)
  o = o + lax.dot_general(h_ref[...], wr_ref[...], (((1,), (1,)), ((), ())),
                          preferred_element_type=jnp.float32)
  o_ref[...] = o + b_ref[...]


def _dense1(acc, dg, x, wl, wr, b):
  grid = (pl.cdiv(N, _BN),)
  return pl.pallas_call(
      _dense1_body,
      grid=grid,
      in_specs=[
          pl.BlockSpec((NC, _BN, D), lambda i: (0, i, 0)),
          pl.BlockSpec((NC, _BN // 8, D), lambda i: (0, i, 0)),
          pl.BlockSpec((_BN, D), lambda i: (i, 0)),
          pl.BlockSpec((D, D), lambda i: (0, 0)),
          pl.BlockSpec((D, D), lambda i: (0, 0)),
          pl.BlockSpec((1, D), lambda i: (0, 0)),
      ],
      out_specs=pl.BlockSpec((_BN, D), lambda i: (i, 0)),
      out_shape=jax.ShapeDtypeStruct((N, D), jnp.float32),
  )(acc, dg, x, wl, wr, b)


def _dense2(acc, dg, h, wl, wr, b):
  grid = (pl.cdiv(N, _BN),)
  return pl.pallas_call(
      _dense2_body,
      grid=grid,
      in_specs=[
          pl.BlockSpec((NC, _BN, D), lambda i: (0, i, 0)),
          pl.BlockSpec((NC, _BN // 8, D), lambda i: (0, i, 0)),
          pl.BlockSpec((_BN, D), lambda i: (i, 0)),
          pl.BlockSpec((D, D), lambda i: (0, 0)),
          pl.BlockSpec((D, D), lambda i: (0, 0)),
          pl.BlockSpec((1, D), lambda i: (0, 0)),
      ],
      out_specs=pl.BlockSpec((_BN, D), lambda i: (i, 0)),
      out_shape=jax.ShapeDtypeStruct((N, D), jnp.float32),
  )(acc, dg, h, wl, wr, b)


_ZEROS_D = np.zeros((RPT, D), np.float32)
_ZEROS_DG = np.zeros((RPT, DG), np.float32)
_ONES_C = np.ones((C, DG), np.float32)


def kernel(x, edge_index, W1_l, W1_r, b1, W2_l, W2_r, b2):
  idx5 = edge_index.reshape(2, NW, NB, SB, C)
  acc1, dg = _segsum_l1(x, idx5, _ZEROS_D, _ONES_C, _ZEROS_DG)
  dgb = dg.reshape(NC, NPAD // 8, 8 * DG)
  h = _dense1(acc1, dgb, x, W1_l, W1_r, b1.reshape(1, D))
  acc2 = _segsum_l2(h, idx5, _ZEROS_D)
  out = _dense2(acc2, dgb, h, W2_l, W2_r, b2.reshape(1, D))
  return out
